# baseline (speedup 1.0000x reference)
import functools

import jax
import jax.numpy as jnp
import numpy as np
from jax import lax
from jax.experimental import pallas as pl
from jax.experimental.pallas import tpu as pltpu
from jax.experimental.pallas import tpu_sc as plsc

_GAMMA = 12.0
_B = 16384
_D = 128
_L = 16
_NC, _NS = 2, 16
_NW = _NC * _NS
_BPW = _B // _NW
_CHUNK = 128
_NCHUNK = _BPW // _CHUNK
_DV = _D // _L

_mesh = plsc.VectorSubcoreMesh(core_axis_name="c", subcore_axis_name="s")


@functools.partial(
    pl.kernel,
    out_type=jax.ShapeDtypeStruct((_B * _L,), jnp.float32),
    mesh=_mesh,
    scratch_types=[
        pltpu.VMEM((_BPW,), jnp.int32),
        pltpu.VMEM((_BPW,), jnp.int32),
        pltpu.VMEM((_BPW,), jnp.int32),
        pltpu.VMEM((2, _CHUNK, _D), jnp.float32),
        pltpu.VMEM((2, _CHUNK, _D), jnp.float32),
        pltpu.VMEM((2, _CHUNK, _D), jnp.float32),
        pltpu.VMEM((2, _CHUNK * _L), jnp.float32),
        pltpu.SemaphoreType.DMA,
        pltpu.SemaphoreType.DMA,
        pltpu.SemaphoreType.DMA,
    ],
)
def _transe_sc(hi_hbm, ri_hbm, ti_hbm, ent_hbm, rel_hbm, out_hbm,
               hi_v, ri_v, ti_v, h_v, r_v, t_v, acc_v, sem0, sem1, sem_out):
    wid = lax.axis_index("s") * _NC + lax.axis_index("c")
    base = wid * _BPW

    pltpu.sync_copy(hi_hbm.at[pl.ds(base, _BPW)], hi_v)
    pltpu.sync_copy(ri_hbm.at[pl.ds(base, _BPW)], ri_v)
    pltpu.sync_copy(ti_hbm.at[pl.ds(base, _BPW)], ti_v)

    sems = (sem0, sem1)

    def start_gathers(c, slot):
        off = c * _CHUNK
        sem = sems[slot]
        d0 = pltpu.async_copy(ent_hbm.at[hi_v.at[pl.ds(off, _CHUNK)]],
                              h_v.at[slot], sem)
        d1 = pltpu.async_copy(rel_hbm.at[ri_v.at[pl.ds(off, _CHUNK)]],
                              r_v.at[slot], sem)
        d2 = pltpu.async_copy(ent_hbm.at[ti_v.at[pl.ds(off, _CHUNK)]],
                              t_v.at[slot], sem)
        return (d0, d1, d2)

    def compute_chunk(slot):
        hs, rs, ts = h_v.at[slot], r_v.at[slot], t_v.at[slot]
        accs = acc_v.at[slot]

        def body(i, _):
            acc = jnp.zeros((_L,), jnp.float32)
            for j in range(_DV):
                dsl = pl.ds(j * _L, _L)
                acc = acc + jnp.abs(hs[i, dsl] + rs[i, dsl] - ts[i, dsl])
            accs[pl.ds(i * _L, _L)] = acc
            return 0

        lax.fori_loop(0, _CHUNK, body, 0)

    out_pending = None
    pending = start_gathers(0, 0)
    for c in range(_NCHUNK):
        for d in pending:
            d.wait()
        if c + 1 < _NCHUNK:
            pending = start_gathers(c + 1, (c + 1) % 2)
        if out_pending is not None:
            out_pending.wait()
        compute_chunk(c % 2)
        out_pending = pltpu.async_copy(
            acc_v.at[c % 2],
            out_hbm.at[pl.ds((base + c * _CHUNK) * _L, _CHUNK * _L)],
            sem_out)
    out_pending.wait()


_FOLD = np.zeros((_D, _D // _L), np.float32)
for _k in range(_D // _L):
    _FOLD[_k * _L:(_k + 1) * _L, _k] = 1.0

_TC_ROWS = _B * _L // _D
_TC_BLK = 256


def _tc_reduce(p_ref, f_ref, o_ref):
    o_ref[...] = _GAMMA - jnp.dot(p_ref[...], f_ref[...],
                                  preferred_element_type=jnp.float32,
                                  precision=lax.Precision.HIGHEST)


_tc_call = pl.pallas_call(
    _tc_reduce,
    grid=(_TC_ROWS // _TC_BLK,),
    in_specs=[
        pl.BlockSpec((_TC_BLK, _D), lambda i: (i, 0)),
        pl.BlockSpec((_D, _D // _L), lambda i: (0, 0)),
    ],
    out_specs=pl.BlockSpec((_TC_BLK, _D // _L), lambda i: (i, 0)),
    out_shape=jax.ShapeDtypeStruct((_TC_ROWS, _D // _L), jnp.float32),
)


def kernel(sample, entity_embedding, relation_embedding):
    hi = sample[:, 0]
    ri = sample[:, 1]
    ti = sample[:, 2]
    partials = _transe_sc(hi, ri, ti, entity_embedding, relation_embedding)
    scores = _tc_call(partials.reshape(_TC_ROWS, _D), _FOLD)
    return scores.reshape(_B, 1)


# speedup vs baseline: 1.0663x; 1.0663x over previous
import functools

import jax
import jax.numpy as jnp
import numpy as np
from jax import lax
from jax.experimental import pallas as pl
from jax.experimental.pallas import tpu as pltpu
from jax.experimental.pallas import tpu_sc as plsc

_GAMMA = 12.0
_B = 16384
_D = 128
_L = 16
_NC, _NS = 2, 16
_NW = _NC * _NS
_BPW = _B // _NW
_CHUNK = 128
_NCHUNK = _BPW // _CHUNK
_DV = _D // _L

_mesh = plsc.VectorSubcoreMesh(core_axis_name="c", subcore_axis_name="s")


@functools.partial(
    pl.kernel,
    out_type=jax.ShapeDtypeStruct((_B * _L,), jnp.float32),
    mesh=_mesh,
    scratch_types=[
        pltpu.VMEM((_BPW,), jnp.int32),
        pltpu.VMEM((_BPW,), jnp.int32),
        pltpu.VMEM((_BPW,), jnp.int32),
        pltpu.VMEM((2, _CHUNK, _D), jnp.float32),
        pltpu.VMEM((2, _CHUNK, _D), jnp.float32),
        pltpu.VMEM((2, _CHUNK, _D), jnp.float32),
        pltpu.VMEM((2, _CHUNK * _L), jnp.float32),
        pltpu.SemaphoreType.DMA,
        pltpu.SemaphoreType.DMA,
        pltpu.SemaphoreType.DMA,
    ],
)
def _transe_sc(hi_hbm, ri_hbm, ti_hbm, ent_hbm, rel_hbm, out_hbm,
               hi_v, ri_v, ti_v, h_v, r_v, t_v, acc_v, sem0, sem1, sem_out):
    wid = lax.axis_index("s") * _NC + lax.axis_index("c")
    base = wid * _BPW

    pltpu.sync_copy(hi_hbm.at[pl.ds(base, _BPW)], hi_v)
    pltpu.sync_copy(ri_hbm.at[pl.ds(base, _BPW)], ri_v)
    pltpu.sync_copy(ti_hbm.at[pl.ds(base, _BPW)], ti_v)

    sems = (sem0, sem1)

    def start_gathers(c, slot):
        off = c * _CHUNK
        sem = sems[slot]
        d0 = pltpu.async_copy(ent_hbm.at[hi_v.at[pl.ds(off, _CHUNK)]],
                              h_v.at[slot], sem)
        d1 = pltpu.async_copy(rel_hbm.at[ri_v.at[pl.ds(off, _CHUNK)]],
                              r_v.at[slot], sem)
        d2 = pltpu.async_copy(ent_hbm.at[ti_v.at[pl.ds(off, _CHUNK)]],
                              t_v.at[slot], sem)
        return (d0, d1, d2)

    def compute_chunk(slot):
        hs, rs, ts = h_v.at[slot], r_v.at[slot], t_v.at[slot]
        accs = acc_v.at[slot]

        @plsc.parallel_loop(0, _CHUNK, unroll=4)
        def body(i):
            acc = jnp.zeros((_L,), jnp.float32)
            for j in range(_DV):
                dsl = pl.ds(j * _L, _L)
                acc = acc + jnp.abs(hs[i, dsl] + rs[i, dsl] - ts[i, dsl])
            accs[pl.ds(i * _L, _L)] = acc

    out_pending = None
    pending = start_gathers(0, 0)
    for c in range(_NCHUNK):
        for d in pending:
            d.wait()
        if c + 1 < _NCHUNK:
            pending = start_gathers(c + 1, (c + 1) % 2)
        if out_pending is not None:
            out_pending.wait()
        compute_chunk(c % 2)
        out_pending = pltpu.async_copy(
            acc_v.at[c % 2],
            out_hbm.at[pl.ds((base + c * _CHUNK) * _L, _CHUNK * _L)],
            sem_out)
    out_pending.wait()


_FOLD = np.zeros((_D, _D // _L), np.float32)
for _k in range(_D // _L):
    _FOLD[_k * _L:(_k + 1) * _L, _k] = 1.0

_TC_ROWS = _B * _L // _D
_TC_BLK = 2048


def _tc_reduce(p_ref, f_ref, o_ref):
    o_ref[...] = _GAMMA - jnp.dot(p_ref[...], f_ref[...],
                                  preferred_element_type=jnp.float32,
                                  precision=lax.Precision.HIGHEST)


_tc_call = pl.pallas_call(
    _tc_reduce,
    grid=(_TC_ROWS // _TC_BLK,),
    in_specs=[
        pl.BlockSpec((_TC_BLK, _D), lambda i: (i, 0)),
        pl.BlockSpec((_D, _D // _L), lambda i: (0, 0)),
    ],
    out_specs=pl.BlockSpec((_TC_BLK, _D // _L), lambda i: (i, 0)),
    out_shape=jax.ShapeDtypeStruct((_TC_ROWS, _D // _L), jnp.float32),
)


def kernel(sample, entity_embedding, relation_embedding):
    hi = sample[:, 0]
    ri = sample[:, 1]
    ti = sample[:, 2]
    partials = _transe_sc(hi, ri, ti, entity_embedding, relation_embedding)
    scores = _tc_call(partials.reshape(_TC_ROWS, _D), _FOLD)
    return scores.reshape(_B, 1)
